# baseline (speedup 1.0000x reference)
import functools

import jax
import jax.numpy as jnp
from jax import lax
from jax.experimental import pallas as pl
from jax.experimental.pallas import tpu as pltpu
from jax.experimental.pallas import tpu_sc as plsc

_R2 = 0.25 * 0.25
_NS = 32
_B = 16
_N = 4096
_NQ = 1024
_C = 128
_L = 16
_NCHUNK = _N // _L
_K = 16
_H = 512
_FLATH = _H * _NS
_RPS = 8
_SLABB = _RPS * _H
_NSLAB = _NS // _RPS
_CPW = 8

_MESH = plsc.VectorSubcoreMesh(core_axis_name="c", subcore_axis_name="s")
_PARAMS = pltpu.CompilerParams(needs_layout_passes=False)


def _lane0(v):
    return lax.squeeze(lax.slice(v, (0,), (1,)), (0,))


@functools.partial(
    pl.kernel,
    out_type=(
        jax.ShapeDtypeStruct((_B * 2 * _FLATH,), jnp.int32),
        jax.ShapeDtypeStruct((_B, _C + 3, _NS, _NQ), jnp.float32),
    ),
    mesh=_MESH,
    compiler_params=_PARAMS,
    scratch_types=[
        pltpu.VMEM((3 * _N,), jnp.float32),
        pltpu.VMEM((3 * _NQ,), jnp.float32),
        pltpu.VMEM((_NS + _K * _L,), jnp.int32),
        pltpu.VMEM((_FLATH,), jnp.int32),
        pltpu.VMEM((_CPW * _N,), jnp.float32),
        pltpu.VMEM((2 * _CPW, _RPS, _H), jnp.float32),
        pltpu.VMEM((_H,), jnp.float32),
        pltpu.SemaphoreType.DMA,
        pltpu.SemaphoreType.DMA,
    ],
)
def _qag_kernel(xyz_f_hbm, new_f_hbm, feat_hbm, idx_hbm, out_hbm,
                xt, ct, qb, idxb, fr, ob, cb, in_sem, out_sem):
    s = lax.axis_index("s")
    h = lax.axis_index("c")
    q0 = h * _H
    b0 = s

    iota = lax.iota(jnp.int32, _L)
    zeros = jnp.zeros((_L,), jnp.int32)

    pltpu.sync_copy(xyz_f_hbm.at[b0], xt)
    pltpu.sync_copy(new_f_hbm.at[b0], ct)

    def per_query(qi, carry):
        qsplat = jnp.full((_L,), q0 + qi, jnp.int32)
        cx = plsc.load_gather(ct, [qsplat])
        cy = plsc.load_gather(ct, [qsplat + _NQ])
        cz = plsc.load_gather(ct, [qsplat + 2 * _NQ])

        def cond(jc):
            j, cnt = jc
            return (cnt < _NS) & (j < _NCHUNK // _K)

        def body(jc):
            j, cnt = jc
            for u in range(_K):
                base = (j * _K + u) * _L
                px = xt[pl.ds(base, _L)]
                py = xt[pl.ds(base + _N, _L)]
                pz = xt[pl.ds(base + 2 * _N, _L)]
                dx = px - cx
                dy = py - cy
                dz = pz - cz
                d2 = dx * dx + dy * dy + dz * dz
                m = d2 < _R2
                plsc.store_compressed(qb.at[pl.ds(cnt, _L)], base + iota, mask=m)
                pc = plsc.all_reduce_population_count(m)
                cnt = cnt + (pc if pc.ndim == 0 else _lane0(pc))
            return j + 1, cnt

        _, cnt = lax.while_loop(cond, body, (jnp.int32(0), jnp.int32(0)))
        cntv = jnp.full((_L,), jnp.minimum(cnt, _NS), jnp.int32)
        fv = plsc.load_gather(qb, [zeros])
        fv = jnp.where(jnp.full((_L,), cnt, jnp.int32) > 0, fv, zeros)
        for sslot in range(2):
            cur = qb[pl.ds(sslot * _L, _L)]
            vals = jnp.where(sslot * _L + iota < cntv, cur, fv)
            plsc.store_scatter(idxb, [(sslot * _L + iota) * _H + qi], vals)
        return carry

    lax.fori_loop(0, _H, per_query, 0)
    pltpu.sync_copy(idxb, idx_hbm.at[pl.ds((b0 * 2 + h) * _FLATH, _FLATH)])

    for d in range(3):
        pltpu.sync_copy(xyz_f_hbm.at[b0, pl.ds(d * _N, _N)], fr.at[pl.ds(0, _N)])
        pltpu.sync_copy(new_f_hbm.at[b0, pl.ds(d * _NQ + q0, _H)], cb)

        for sl in range(_NSLAB):
            def xyz_chunks(k, carry, sl=sl):
                for u in range(8):
                    r = k * 8 + u
                    iv = idxb[pl.ds(sl * _SLABB + r * _L, _L)]
                    vals = plsc.load_gather(fr, [iv])
                    col = (r & (_H // _L - 1)) * _L
                    cv = cb[pl.ds(col, _L)]
                    ob[r >> 5, 0, pl.ds(col, _L)] = vals - cv
                return carry

            lax.fori_loop(0, _SLABB // (8 * _L), xyz_chunks, 0)
            pltpu.sync_copy(
                ob.at[pl.ds(0, _RPS), 0, :],
                out_hbm.at[b0, _C + d, pl.ds(sl * _RPS, _RPS), pl.ds(q0, _H)])

    plsc.subcore_barrier()

    c0 = s * _CPW

    def _issue_in(b):
        pltpu.async_copy(idx_hbm.at[pl.ds((b * 2 + h) * _FLATH, _FLATH)],
                         idxb, in_sem)
        for ci in range(_CPW):
            pltpu.async_copy(feat_hbm.at[b, c0 + ci],
                             fr.at[pl.ds(ci * _N, _N)], in_sem)

    _issue_in(jnp.int32(0))

    def per_batch(b, carry):
        pltpu.make_async_copy(
            idx_hbm.at[pl.ds((b * 2 + h) * _FLATH, _FLATH)], idxb, in_sem).wait()
        for ci in range(_CPW):
            pltpu.make_async_copy(feat_hbm.at[b, c0 + ci],
                                  fr.at[pl.ds(ci * _N, _N)], in_sem).wait()

        hnds = []
        for sl in range(_NSLAB):
            pb = sl & 1
            if sl >= 2:
                hnds[sl - 2].wait()

            def gather_chunks(k, inner, sl=sl, pb=pb):
                for u in range(8):
                    r = k * 8 + u
                    iv = idxb[pl.ds(sl * _SLABB + r * _L, _L)]
                    sr = r >> 5
                    col = (r & (_H // _L - 1)) * _L
                    for ci in range(_CPW):
                        vals = plsc.load_gather(fr, [iv + ci * _N])
                        ob[pb * _CPW + ci, sr, pl.ds(col, _L)] = vals
                return inner

            lax.fori_loop(0, _SLABB // (8 * _L), gather_chunks, 0)
            if sl == _NSLAB - 1:
                @pl.when(b + 1 < _B)
                def _(b=b):
                    _issue_in(b + 1)
            hnds.append(pltpu.async_copy(
                ob.at[pl.ds(pb * _CPW, _CPW)],
                out_hbm.at[b, pl.ds(c0, _CPW), pl.ds(sl * _RPS, _RPS),
                           pl.ds(q0, _H)],
                out_sem))

        for x in hnds[_NSLAB - 2:]:
            x.wait()
        return carry

    lax.fori_loop(0, _B, per_batch, 0)


def kernel(xyz, new_xyz, features):
    xyz_f = jnp.transpose(xyz, (0, 2, 1)).reshape(_B, 3 * _N)
    new_f = jnp.transpose(new_xyz, (0, 2, 1)).reshape(_B, 3 * _NQ)
    _, out = _qag_kernel(xyz_f, new_f, features)
    return jnp.transpose(out, (0, 1, 3, 2))


# speedup vs baseline: 1.0079x; 1.0079x over previous
import functools

import jax
import jax.numpy as jnp
from jax import lax
from jax.experimental import pallas as pl
from jax.experimental.pallas import tpu as pltpu
from jax.experimental.pallas import tpu_sc as plsc

_R2 = 0.25 * 0.25
_NS = 32
_B = 16
_N = 4096
_NQ = 1024
_C = 128
_L = 16
_NCHUNK = _N // _L
_K = 8
_H = 512
_FLATH = _H * _NS
_RPS = 8
_SLABB = _RPS * _H
_NSLAB = _NS // _RPS
_CPW = 8

_MESH = plsc.VectorSubcoreMesh(core_axis_name="c", subcore_axis_name="s")
_PARAMS = pltpu.CompilerParams(needs_layout_passes=False)


def _lane0(v):
    return lax.squeeze(lax.slice(v, (0,), (1,)), (0,))


@functools.partial(
    pl.kernel,
    out_type=(
        jax.ShapeDtypeStruct((_B * 2 * _FLATH,), jnp.int32),
        jax.ShapeDtypeStruct((_B, _C + 3, _NS, _NQ), jnp.float32),
    ),
    mesh=_MESH,
    compiler_params=_PARAMS,
    scratch_types=[
        pltpu.VMEM((3 * _N,), jnp.float32),
        pltpu.VMEM((3 * _NQ,), jnp.float32),
        pltpu.VMEM((_NS + _K * _L,), jnp.int32),
        pltpu.VMEM((_FLATH,), jnp.int32),
        pltpu.VMEM((_CPW * _N,), jnp.float32),
        pltpu.VMEM((2 * _CPW, _RPS, _H), jnp.float32),
        pltpu.VMEM((_H,), jnp.float32),
        pltpu.SemaphoreType.DMA,
        pltpu.SemaphoreType.DMA,
    ],
)
def _qag_kernel(xyz_f_hbm, new_f_hbm, feat_hbm, idx_hbm, out_hbm,
                xt, ct, qb, idxb, fr, ob, cb, in_sem, out_sem):
    s = lax.axis_index("s")
    h = lax.axis_index("c")
    q0 = h * _H
    b0 = s

    iota = lax.iota(jnp.int32, _L)
    zeros = jnp.zeros((_L,), jnp.int32)

    pltpu.sync_copy(xyz_f_hbm.at[b0], xt)
    pltpu.sync_copy(new_f_hbm.at[b0], ct)

    def per_query(qi, carry):
        qsplat = jnp.full((_L,), q0 + qi, jnp.int32)
        cx = plsc.load_gather(ct, [qsplat])
        cy = plsc.load_gather(ct, [qsplat + _NQ])
        cz = plsc.load_gather(ct, [qsplat + 2 * _NQ])

        def cond(jc):
            j, cnt = jc
            return (cnt < _NS) & (j < _NCHUNK // _K)

        def body(jc):
            j, cnt = jc
            for u in range(_K):
                base = (j * _K + u) * _L
                px = xt[pl.ds(base, _L)]
                py = xt[pl.ds(base + _N, _L)]
                pz = xt[pl.ds(base + 2 * _N, _L)]
                dx = px - cx
                dy = py - cy
                dz = pz - cz
                d2 = dx * dx + dy * dy + dz * dz
                m = d2 < _R2
                plsc.store_compressed(qb.at[pl.ds(cnt, _L)], base + iota, mask=m)
                pc = plsc.all_reduce_population_count(m)
                cnt = cnt + (pc if pc.ndim == 0 else _lane0(pc))
            return j + 1, cnt

        _, cnt = lax.while_loop(cond, body, (jnp.int32(0), jnp.int32(0)))
        cntv = jnp.full((_L,), jnp.minimum(cnt, _NS), jnp.int32)
        fv = plsc.load_gather(qb, [zeros])
        fv = jnp.where(jnp.full((_L,), cnt, jnp.int32) > 0, fv, zeros)
        for sslot in range(2):
            cur = qb[pl.ds(sslot * _L, _L)]
            vals = jnp.where(sslot * _L + iota < cntv, cur, fv)
            plsc.store_scatter(idxb, [(sslot * _L + iota) * _H + qi], vals)
        return carry

    lax.fori_loop(0, _H, per_query, 0)
    pltpu.sync_copy(idxb, idx_hbm.at[pl.ds((b0 * 2 + h) * _FLATH, _FLATH)])

    for d in range(3):
        pltpu.sync_copy(xyz_f_hbm.at[b0, pl.ds(d * _N, _N)], fr.at[pl.ds(0, _N)])
        pltpu.sync_copy(new_f_hbm.at[b0, pl.ds(d * _NQ + q0, _H)], cb)

        for sl in range(_NSLAB):
            def xyz_chunks(k, carry, sl=sl):
                for u in range(8):
                    r = k * 8 + u
                    iv = idxb[pl.ds(sl * _SLABB + r * _L, _L)]
                    vals = plsc.load_gather(fr, [iv])
                    col = (r & (_H // _L - 1)) * _L
                    cv = cb[pl.ds(col, _L)]
                    ob[r >> 5, 0, pl.ds(col, _L)] = vals - cv
                return carry

            lax.fori_loop(0, _SLABB // (8 * _L), xyz_chunks, 0)
            pltpu.sync_copy(
                ob.at[pl.ds(0, _RPS), 0, :],
                out_hbm.at[b0, _C + d, pl.ds(sl * _RPS, _RPS), pl.ds(q0, _H)])

    plsc.subcore_barrier()

    c0 = s * _CPW

    def _issue_in(b):
        pltpu.async_copy(idx_hbm.at[pl.ds((b * 2 + h) * _FLATH, _FLATH)],
                         idxb, in_sem)
        for ci in range(_CPW):
            pltpu.async_copy(feat_hbm.at[b, c0 + ci],
                             fr.at[pl.ds(ci * _N, _N)], in_sem)

    _issue_in(jnp.int32(0))

    def per_batch(b, carry):
        pltpu.make_async_copy(
            idx_hbm.at[pl.ds((b * 2 + h) * _FLATH, _FLATH)], idxb, in_sem).wait()
        for ci in range(_CPW):
            pltpu.make_async_copy(feat_hbm.at[b, c0 + ci],
                                  fr.at[pl.ds(ci * _N, _N)], in_sem).wait()

        hnds = []
        for sl in range(_NSLAB):
            pb = sl & 1
            if sl >= 2:
                hnds[sl - 2].wait()

            def gather_chunks(k, inner, sl=sl, pb=pb):
                for u in range(8):
                    r = k * 8 + u
                    iv = idxb[pl.ds(sl * _SLABB + r * _L, _L)]
                    sr = r >> 5
                    col = (r & (_H // _L - 1)) * _L
                    for ci in range(_CPW):
                        vals = plsc.load_gather(fr, [iv + ci * _N])
                        ob[pb * _CPW + ci, sr, pl.ds(col, _L)] = vals
                return inner

            lax.fori_loop(0, _SLABB // (8 * _L), gather_chunks, 0)
            if sl == _NSLAB - 1:
                @pl.when(b + 1 < _B)
                def _(b=b):
                    _issue_in(b + 1)
            hnds.append(pltpu.async_copy(
                ob.at[pl.ds(pb * _CPW, _CPW)],
                out_hbm.at[b, pl.ds(c0, _CPW), pl.ds(sl * _RPS, _RPS),
                           pl.ds(q0, _H)],
                out_sem))

        for x in hnds[_NSLAB - 2:]:
            x.wait()
        return carry

    lax.fori_loop(0, _B, per_batch, 0)


def kernel(xyz, new_xyz, features):
    xyz_f = jnp.transpose(xyz, (0, 2, 1)).reshape(_B, 3 * _N)
    new_f = jnp.transpose(new_xyz, (0, 2, 1)).reshape(_B, 3 * _NQ)
    _, out = _qag_kernel(xyz_f, new_f, features)
    return jnp.transpose(out, (0, 1, 3, 2))


# speedup vs baseline: 1.2615x; 1.2516x over previous
import functools

import jax
import jax.numpy as jnp
from jax import lax
from jax.experimental import pallas as pl
from jax.experimental.pallas import tpu as pltpu
from jax.experimental.pallas import tpu_sc as plsc

_R2 = 0.25 * 0.25
_NS = 32
_B = 16
_N = 4096
_NQ = 1024
_C = 128
_L = 16
_NCHUNK = _N // _L
_K = 8
_H = 512
_FLATH = _H * _NS
_RPS = 8
_SLABB = _RPS * _H
_NSLAB = _NS // _RPS
_CPW = 8

_MESH = plsc.VectorSubcoreMesh(core_axis_name="c", subcore_axis_name="s")
_PARAMS = pltpu.CompilerParams(needs_layout_passes=False)


def _lane0(v):
    return lax.squeeze(lax.slice(v, (0,), (1,)), (0,))


@functools.partial(
    pl.kernel,
    out_type=(
        jax.ShapeDtypeStruct((_B * 2 * _FLATH,), jnp.int32),
        jax.ShapeDtypeStruct((_B, _C + 3, _NS, _NQ), jnp.float32),
    ),
    mesh=_MESH,
    compiler_params=_PARAMS,
    scratch_types=[
        pltpu.VMEM((3 * _N,), jnp.float32),
        pltpu.VMEM((3 * _NQ,), jnp.float32),
        pltpu.VMEM((_NS + _K * _L,), jnp.int32),
        pltpu.VMEM((_FLATH,), jnp.int32),
        pltpu.VMEM((_CPW * _N,), jnp.float32),
        pltpu.VMEM((2 * _CPW, _RPS, _H), jnp.float32),
        pltpu.VMEM((_H,), jnp.float32),
        pltpu.SemaphoreType.DMA,
        pltpu.SemaphoreType.DMA,
    ],
)
def _qag_kernel(xyz_f_hbm, new_f_hbm, feat_hbm, idx_hbm, out_hbm,
                xt, ct, qb, idxb, fr, ob, cb, in_sem, out_sem):
    s = lax.axis_index("s")
    h = lax.axis_index("c")
    q0 = h * _H
    b0 = s

    iota = lax.iota(jnp.int32, _L)
    zeros = jnp.zeros((_L,), jnp.int32)

    pltpu.sync_copy(xyz_f_hbm.at[b0], xt)
    pltpu.sync_copy(new_f_hbm.at[b0], ct)

    def per_query(qi, carry):
        qsplat = jnp.full((_L,), q0 + qi, jnp.int32)
        cx = plsc.load_gather(ct, [qsplat])
        cy = plsc.load_gather(ct, [qsplat + _NQ])
        cz = plsc.load_gather(ct, [qsplat + 2 * _NQ])

        def cond(jc):
            j, cnt = jc
            return (cnt < _NS) & (j < _NCHUNK // _K)

        def body(jc):
            j, cnt = jc
            masks = []
            pcs = []
            for u in range(_K):
                base = (j * _K + u) * _L
                px = xt[pl.ds(base, _L)]
                py = xt[pl.ds(base + _N, _L)]
                pz = xt[pl.ds(base + 2 * _N, _L)]
                dx = px - cx
                dy = py - cy
                dz = pz - cz
                d2 = dx * dx + dy * dy + dz * dz
                m = d2 < _R2
                masks.append(m)
                pc = plsc.all_reduce_population_count(m)
                pcs.append(pc if pc.ndim == 0 else _lane0(pc))
            for u in range(_K):
                base = (j * _K + u) * _L
                plsc.store_compressed(qb.at[pl.ds(cnt, _L)], base + iota,
                                      mask=masks[u])
                cnt = cnt + pcs[u]
            return j + 1, cnt

        _, cnt = lax.while_loop(cond, body, (jnp.int32(0), jnp.int32(0)))
        cntv = jnp.full((_L,), jnp.minimum(cnt, _NS), jnp.int32)
        fv = plsc.load_gather(qb, [zeros])
        fv = jnp.where(jnp.full((_L,), cnt, jnp.int32) > 0, fv, zeros)
        for sslot in range(2):
            cur = qb[pl.ds(sslot * _L, _L)]
            vals = jnp.where(sslot * _L + iota < cntv, cur, fv)
            plsc.store_scatter(idxb, [(sslot * _L + iota) * _H + qi], vals)
        return carry

    lax.fori_loop(0, _H, per_query, 0)
    pltpu.sync_copy(idxb, idx_hbm.at[pl.ds((b0 * 2 + h) * _FLATH, _FLATH)])

    for d in range(3):
        pltpu.sync_copy(xyz_f_hbm.at[b0, pl.ds(d * _N, _N)], fr.at[pl.ds(0, _N)])
        pltpu.sync_copy(new_f_hbm.at[b0, pl.ds(d * _NQ + q0, _H)], cb)

        for sl in range(_NSLAB):
            def xyz_chunks(k, carry, sl=sl):
                for u in range(8):
                    r = k * 8 + u
                    iv = idxb[pl.ds(sl * _SLABB + r * _L, _L)]
                    vals = plsc.load_gather(fr, [iv])
                    col = (r & (_H // _L - 1)) * _L
                    cv = cb[pl.ds(col, _L)]
                    ob[r >> 5, 0, pl.ds(col, _L)] = vals - cv
                return carry

            lax.fori_loop(0, _SLABB // (8 * _L), xyz_chunks, 0)
            pltpu.sync_copy(
                ob.at[pl.ds(0, _RPS), 0, :],
                out_hbm.at[b0, _C + d, pl.ds(sl * _RPS, _RPS), pl.ds(q0, _H)])

    plsc.subcore_barrier()

    c0 = s * _CPW

    def _issue_in(b):
        pltpu.async_copy(idx_hbm.at[pl.ds((b * 2 + h) * _FLATH, _FLATH)],
                         idxb, in_sem)
        for ci in range(_CPW):
            pltpu.async_copy(feat_hbm.at[b, c0 + ci],
                             fr.at[pl.ds(ci * _N, _N)], in_sem)

    _issue_in(jnp.int32(0))

    def per_batch(b, carry):
        pltpu.make_async_copy(
            idx_hbm.at[pl.ds((b * 2 + h) * _FLATH, _FLATH)], idxb, in_sem).wait()
        for ci in range(_CPW):
            pltpu.make_async_copy(feat_hbm.at[b, c0 + ci],
                                  fr.at[pl.ds(ci * _N, _N)], in_sem).wait()

        hnds = []
        for sl in range(_NSLAB):
            pb = sl & 1
            if sl >= 2:
                hnds[sl - 2].wait()

            def gather_chunks(k, inner, sl=sl, pb=pb):
                for u in range(8):
                    r = k * 8 + u
                    iv = idxb[pl.ds(sl * _SLABB + r * _L, _L)]
                    sr = r >> 5
                    col = (r & (_H // _L - 1)) * _L
                    for ci in range(_CPW):
                        vals = plsc.load_gather(fr, [iv + ci * _N])
                        ob[pb * _CPW + ci, sr, pl.ds(col, _L)] = vals
                return inner

            lax.fori_loop(0, _SLABB // (8 * _L), gather_chunks, 0)
            if sl == _NSLAB - 1:
                @pl.when(b + 1 < _B)
                def _(b=b):
                    _issue_in(b + 1)
            hnds.append(pltpu.async_copy(
                ob.at[pl.ds(pb * _CPW, _CPW)],
                out_hbm.at[b, pl.ds(c0, _CPW), pl.ds(sl * _RPS, _RPS),
                           pl.ds(q0, _H)],
                out_sem))

        for x in hnds[_NSLAB - 2:]:
            x.wait()
        return carry

    lax.fori_loop(0, _B, per_batch, 0)


def kernel(xyz, new_xyz, features):
    xyz_f = jnp.transpose(xyz, (0, 2, 1)).reshape(_B, 3 * _N)
    new_f = jnp.transpose(new_xyz, (0, 2, 1)).reshape(_B, 3 * _NQ)
    _, out = _qag_kernel(xyz_f, new_f, features)
    return jnp.transpose(out, (0, 1, 3, 2))


# speedup vs baseline: 1.2709x; 1.0074x over previous
import functools

import jax
import jax.numpy as jnp
from jax import lax
from jax.experimental import pallas as pl
from jax.experimental.pallas import tpu as pltpu
from jax.experimental.pallas import tpu_sc as plsc

_R2 = 0.25 * 0.25
_NS = 32
_B = 16
_N = 4096
_NQ = 1024
_C = 128
_L = 16
_NCHUNK = _N // _L
_K = 8
_H = 512
_FLATH = _H * _NS
_RPS = 8
_SLABB = _RPS * _H
_NSLAB = _NS // _RPS
_CPW = 8

_MESH = plsc.VectorSubcoreMesh(core_axis_name="c", subcore_axis_name="s")
_PARAMS = pltpu.CompilerParams(needs_layout_passes=False)


def _lane0(v):
    return lax.squeeze(lax.slice(v, (0,), (1,)), (0,))


@functools.partial(
    pl.kernel,
    out_type=(
        jax.ShapeDtypeStruct((_B * 2 * _FLATH,), jnp.int32),
        jax.ShapeDtypeStruct((_B, _C + 3, _NS, _NQ), jnp.float32),
    ),
    mesh=_MESH,
    compiler_params=_PARAMS,
    scratch_types=[
        pltpu.VMEM((3 * _N,), jnp.float32),
        pltpu.VMEM((3 * _NQ,), jnp.float32),
        pltpu.VMEM((_NS + _K * _L,), jnp.int32),
        pltpu.VMEM((_FLATH,), jnp.int32),
        pltpu.VMEM((_CPW * _N,), jnp.float32),
        pltpu.VMEM((2 * _CPW, _RPS, _H), jnp.float32),
        pltpu.VMEM((_H,), jnp.float32),
        pltpu.SemaphoreType.DMA,
        pltpu.SemaphoreType.DMA,
    ],
)
def _qag_kernel(xyz_f_hbm, new_f_hbm, feat_hbm, idx_hbm, out_hbm,
                xt, ct, qb, idxb, fr, ob, cb, in_sem, out_sem):
    s = lax.axis_index("s")
    h = lax.axis_index("c")
    q0 = h * _H
    b0 = s

    iota = lax.iota(jnp.int32, _L)
    zeros = jnp.zeros((_L,), jnp.int32)

    pltpu.sync_copy(xyz_f_hbm.at[b0], xt)
    pltpu.sync_copy(new_f_hbm.at[b0], ct)

    def per_query(qi, carry):
        qsplat = jnp.full((_L,), q0 + qi, jnp.int32)
        cx = plsc.load_gather(ct, [qsplat])
        cy = plsc.load_gather(ct, [qsplat + _NQ])
        cz = plsc.load_gather(ct, [qsplat + 2 * _NQ])

        def cond(jc):
            j, cnt = jc
            return (cnt < _NS) & (j < _NCHUNK // _K)

        def body(jc):
            j, cnt = jc
            masks = []
            pcs = []
            for u in range(_K):
                base = (j * _K + u) * _L
                px = xt[pl.ds(base, _L)]
                py = xt[pl.ds(base + _N, _L)]
                pz = xt[pl.ds(base + 2 * _N, _L)]
                dx = px - cx
                dy = py - cy
                dz = pz - cz
                d2 = dx * dx + dy * dy + dz * dz
                m = d2 < _R2
                masks.append(m)
                pc = plsc.all_reduce_population_count(m)
                pcs.append(pc if pc.ndim == 0 else _lane0(pc))
            for u in range(_K):
                base = (j * _K + u) * _L
                plsc.store_compressed(qb.at[pl.ds(cnt, _L)], base + iota,
                                      mask=masks[u])
                cnt = cnt + pcs[u]
            return j + 1, cnt

        _, cnt = lax.while_loop(cond, body, (jnp.int32(0), jnp.int32(0)))
        cntv = jnp.full((_L,), jnp.minimum(cnt, _NS), jnp.int32)
        fv = plsc.load_gather(qb, [zeros])
        fv = jnp.where(jnp.full((_L,), cnt, jnp.int32) > 0, fv, zeros)
        for sslot in range(2):
            cur = qb[pl.ds(sslot * _L, _L)]
            vals = jnp.where(sslot * _L + iota < cntv, cur, fv)
            plsc.store_scatter(idxb, [(sslot * _L + iota) * _H + qi], vals)
        return carry

    lax.fori_loop(0, _H, per_query, 0)
    pltpu.sync_copy(idxb, idx_hbm.at[pl.ds((b0 * 2 + h) * _FLATH, _FLATH)])

    for d in range(3):
        pltpu.sync_copy(xyz_f_hbm.at[b0, pl.ds(d * _N, _N)], fr.at[pl.ds(0, _N)])
        pltpu.sync_copy(new_f_hbm.at[b0, pl.ds(d * _NQ + q0, _H)], cb)

        for sl in range(_NSLAB):
            def xyz_chunks(k, carry, sl=sl):
                for u in range(8):
                    r = k * 8 + u
                    iv = idxb[pl.ds(sl * _SLABB + r * _L, _L)]
                    vals = plsc.load_gather(fr, [iv])
                    col = (r & (_H // _L - 1)) * _L
                    cv = cb[pl.ds(col, _L)]
                    ob[r >> 5, 0, pl.ds(col, _L)] = vals - cv
                return carry

            lax.fori_loop(0, _SLABB // (8 * _L), xyz_chunks, 0)
            pltpu.sync_copy(
                ob.at[pl.ds(0, _RPS), 0, :],
                out_hbm.at[b0, _C + d, pl.ds(sl * _RPS, _RPS), pl.ds(q0, _H)])

    c0 = s * _CPW

    def _issue_in(b):
        pltpu.async_copy(idx_hbm.at[pl.ds((b * 2 + h) * _FLATH, _FLATH)],
                         idxb, in_sem)
        for ci in range(_CPW):
            pltpu.async_copy(feat_hbm.at[b, c0 + ci],
                             fr.at[pl.ds(ci * _N, _N)], in_sem)

    def per_batch(b, prefetch):
        pltpu.make_async_copy(
            idx_hbm.at[pl.ds((b * 2 + h) * _FLATH, _FLATH)], idxb, in_sem).wait()
        for ci in range(_CPW):
            pltpu.make_async_copy(feat_hbm.at[b, c0 + ci],
                                  fr.at[pl.ds(ci * _N, _N)], in_sem).wait()

        hnds = []
        for sl in range(_NSLAB):
            pb = sl & 1
            if sl >= 2:
                hnds[sl - 2].wait()

            def gather_chunks(k, inner, sl=sl, pb=pb):
                for u in range(8):
                    r = k * 8 + u
                    iv = idxb[pl.ds(sl * _SLABB + r * _L, _L)]
                    sr = r >> 5
                    col = (r & (_H // _L - 1)) * _L
                    for ci in range(_CPW):
                        vals = plsc.load_gather(fr, [iv + ci * _N])
                        ob[pb * _CPW + ci, sr, pl.ds(col, _L)] = vals
                return inner

            lax.fori_loop(0, _SLABB // (8 * _L), gather_chunks, 0)
            if sl == _NSLAB - 1 and prefetch is not None:
                prefetch(b)
            hnds.append(pltpu.async_copy(
                ob.at[pl.ds(pb * _CPW, _CPW)],
                out_hbm.at[b, pl.ds(c0, _CPW), pl.ds(sl * _RPS, _RPS),
                           pl.ds(q0, _H)],
                out_sem))

        for x in hnds[_NSLAB - 2:]:
            x.wait()

    _issue_in(b0)
    per_batch(b0, None)

    plsc.subcore_barrier()

    _issue_in((b0 + 1) & (_B - 1))

    def loop_body(i, carry):
        b = (b0 + i) & (_B - 1)

        def prefetch(b):
            @pl.when(i + 1 < _B)
            def _():
                _issue_in((b + 1) & (_B - 1))

        per_batch(b, prefetch)
        return carry

    lax.fori_loop(1, _B, loop_body, 0)


def kernel(xyz, new_xyz, features):
    xyz_f = jnp.transpose(xyz, (0, 2, 1)).reshape(_B, 3 * _N)
    new_f = jnp.transpose(new_xyz, (0, 2, 1)).reshape(_B, 3 * _NQ)
    _, out = _qag_kernel(xyz_f, new_f, features)
    return jnp.transpose(out, (0, 1, 3, 2))
